# baseline (speedup 1.0000x reference)
import functools

import jax
import jax.numpy as jnp
from jax.experimental import pallas as pl
from jax.experimental.pallas import tpu as pltpu

_R = 5
_BR = 40


def _graph_conv_kernel(*refs):
    a_refs = refs[:_R]
    x_ref, xi_ref, wt_ref, b_ref, o_ref = refs[_R:]
    s = pl.program_id(1)
    x = x_ref[:]
    wt = wt_ref[0]
    for r in range(_R):
        a = a_refs[r][0].astype(jnp.bfloat16)
        p = jnp.dot(a, x, preferred_element_type=jnp.float32)
        xi = xi_ref[r * _BR:(r + 1) * _BR, :]
        t = jnp.dot(p + xi, wt, preferred_element_type=jnp.float32)

        @pl.when(s == 0)
        def _first(r=r, t=t):
            o_ref[r * _BR:(r + 1) * _BR, :] = t + b_ref[:]

        @pl.when(s != 0)
        def _rest(r=r, t=t):
            o_ref[r * _BR:(r + 1) * _BR, :] += t


@jax.jit
def _graph_conv(x0_bf16, x0, supports, wt, b2d):
    n, d = x0.shape
    out = wt.shape[2]
    bn = _R * _BR
    a_specs = [
        pl.BlockSpec((1, _BR, n), functools.partial(
            lambda i, s, r=0: (s, i * _R + r, 0), r=r))
        for r in range(_R)
    ]
    return pl.pallas_call(
        _graph_conv_kernel,
        grid=(n // bn, 2),
        in_specs=a_specs + [
            pl.BlockSpec((n, d), lambda i, s: (0, 0)),
            pl.BlockSpec((bn, d), lambda i, s: (i, 0)),
            pl.BlockSpec((1, d, out), lambda i, s: (s, 0, 0)),
            pl.BlockSpec((1, out), lambda i, s: (0, 0)),
        ],
        out_specs=pl.BlockSpec((bn, out), lambda i, s: (i, 0)),
        out_shape=jax.ShapeDtypeStruct((n, out), jnp.float32),
        compiler_params=pltpu.CompilerParams(
            dimension_semantics=("parallel", "arbitrary"),
        ),
    )(*([supports] * _R), x0_bf16, x0, wt, b2d)


def kernel(inputs, supports, W, b):
    bsz, n, d = inputs.shape
    s = supports.shape[0]
    out_dim = W.shape[0]
    x0 = jnp.transpose(inputs, (1, 2, 0)).reshape(n, d * bsz)
    wt = jnp.stack([jnp.transpose(W[:, i::s]) for i in range(s)])
    b2d = b.reshape(1, out_dim)

    res = _graph_conv(x0.astype(jnp.bfloat16), x0, supports, wt, b2d)
    return res.reshape(bsz, n, out_dim)


# speedup vs baseline: 1.3124x; 1.3124x over previous
import functools

import jax
import jax.numpy as jnp
from jax.experimental import pallas as pl
from jax.experimental.pallas import tpu as pltpu

_R = 1
_BR = 200


def _graph_conv_kernel(*refs):
    a_refs = refs[:_R]
    x_ref, xi_ref, wt_ref, b_ref, o_ref = refs[_R:]
    s = pl.program_id(1)
    x = x_ref[:]
    wt = wt_ref[0]
    for r in range(_R):
        a = a_refs[r][0].astype(jnp.bfloat16)
        p = jnp.dot(a, x, preferred_element_type=jnp.float32)
        xi = xi_ref[r * _BR:(r + 1) * _BR, :]
        t = jnp.dot(p + xi, wt, preferred_element_type=jnp.float32)

        @pl.when(s == 0)
        def _first(r=r, t=t):
            o_ref[r * _BR:(r + 1) * _BR, :] = t + b_ref[:]

        @pl.when(s != 0)
        def _rest(r=r, t=t):
            o_ref[r * _BR:(r + 1) * _BR, :] += t


@jax.jit
def _graph_conv(x0_bf16, x0, supports, wt, b2d):
    n, d = x0.shape
    out = wt.shape[2]
    bn = _R * _BR
    a_specs = [
        pl.BlockSpec((1, _BR, n), functools.partial(
            lambda i, s, r=0: (s, i * _R + r, 0), r=r))
        for r in range(_R)
    ]
    return pl.pallas_call(
        _graph_conv_kernel,
        grid=(n // bn, 2),
        in_specs=a_specs + [
            pl.BlockSpec((n, d), lambda i, s: (0, 0)),
            pl.BlockSpec((bn, d), lambda i, s: (i, 0)),
            pl.BlockSpec((1, d, out), lambda i, s: (s, 0, 0)),
            pl.BlockSpec((1, out), lambda i, s: (0, 0)),
        ],
        out_specs=pl.BlockSpec((bn, out), lambda i, s: (i, 0)),
        out_shape=jax.ShapeDtypeStruct((n, out), jnp.float32),
        compiler_params=pltpu.CompilerParams(
            dimension_semantics=("parallel", "arbitrary"),
        ),
    )(*([supports] * _R), x0_bf16, x0, wt, b2d)


def kernel(inputs, supports, W, b):
    bsz, n, d = inputs.shape
    s = supports.shape[0]
    out_dim = W.shape[0]
    x0 = jnp.transpose(inputs, (1, 2, 0)).reshape(n, d * bsz)
    wt = jnp.stack([jnp.transpose(W[:, i::s]) for i in range(s)])
    b2d = b.reshape(1, out_dim)

    res = _graph_conv(x0.astype(jnp.bfloat16), x0, supports, wt, b2d)
    return res.reshape(bsz, n, out_dim)


# speedup vs baseline: 1.3539x; 1.0317x over previous
import functools

import jax
import jax.numpy as jnp
from jax.experimental import pallas as pl
from jax.experimental.pallas import tpu as pltpu

_R = 2
_BR = 168


def _graph_conv_kernel(*refs):
    a_refs = refs[:2 * _R]
    x_ref, xi_ref, w0t_ref, w1t_ref, b_ref, o_ref = refs[2 * _R:]
    x = x_ref[:]
    w0t = w0t_ref[:]
    w1t = w1t_ref[:]
    bias = b_ref[:]
    for r in range(_R):
        a0 = a_refs[r][0].astype(jnp.bfloat16)
        a1 = a_refs[_R + r][0].astype(jnp.bfloat16)
        p0 = jnp.dot(a0, x, preferred_element_type=jnp.float32)
        p1 = jnp.dot(a1, x, preferred_element_type=jnp.float32)
        xi = xi_ref[r * _BR:(r + 1) * _BR, :]
        o_ref[r * _BR:(r + 1) * _BR, :] = (
            jnp.dot(p0 + xi, w0t, preferred_element_type=jnp.float32)
            + jnp.dot(p1 + xi, w1t, preferred_element_type=jnp.float32)
            + bias
        )


@jax.jit
def _graph_conv(x0_bf16, x0, supports, w0t, w1t, b2d):
    n, d = x0.shape
    out = w0t.shape[1]
    bn = _R * _BR
    a_specs = [
        pl.BlockSpec((1, _BR, n), functools.partial(
            lambda i, s=0, r=0: (s, i * _R + r, 0), s=s, r=r))
        for s in range(2)
        for r in range(_R)
    ]
    return pl.pallas_call(
        _graph_conv_kernel,
        grid=(pl.cdiv(n, bn),),
        in_specs=a_specs + [
            pl.BlockSpec((n, d), lambda i: (0, 0)),
            pl.BlockSpec((bn, d), lambda i: (i, 0)),
            pl.BlockSpec((d, out), lambda i: (0, 0)),
            pl.BlockSpec((d, out), lambda i: (0, 0)),
            pl.BlockSpec((1, out), lambda i: (0, 0)),
        ],
        out_specs=pl.BlockSpec((bn, out), lambda i: (i, 0)),
        out_shape=jax.ShapeDtypeStruct((n, out), jnp.float32),
        compiler_params=pltpu.CompilerParams(
            dimension_semantics=("arbitrary",),
        ),
    )(*([supports] * (2 * _R)), x0_bf16, x0, w0t, w1t, b2d)


def kernel(inputs, supports, W, b):
    bsz, n, d = inputs.shape
    s = supports.shape[0]
    out_dim = W.shape[0]
    x0 = jnp.transpose(inputs, (1, 2, 0)).reshape(n, d * bsz)
    w0t = jnp.transpose(W[:, 0::s])
    w1t = jnp.transpose(W[:, 1::s])
    b2d = b.reshape(1, out_dim)

    res = _graph_conv(x0.astype(jnp.bfloat16), x0, supports, w0t, w1t, b2d)
    return res.reshape(bsz, n, out_dim)


# speedup vs baseline: 1.3831x; 1.0216x over previous
import functools

import jax
import jax.numpy as jnp
from jax.experimental import pallas as pl
from jax.experimental.pallas import tpu as pltpu

_R = 1
_BR = 200


def _graph_conv_kernel(*refs):
    i = pl.program_id(0)
    a_refs = refs[:2 * _R]
    x_ref, xf_ref, w0t_ref, w1t_ref, b_ref, o_ref = refs[2 * _R:]
    x = x_ref[:]
    w0t = w0t_ref[:]
    w1t = w1t_ref[:]
    bias = b_ref[:]
    bn = _R * _BR
    for r in range(_R):
        a0 = a_refs[r][0].astype(jnp.bfloat16)
        a1 = a_refs[_R + r][0].astype(jnp.bfloat16)
        p0 = jnp.dot(a0, x, preferred_element_type=jnp.float32)
        p1 = jnp.dot(a1, x, preferred_element_type=jnp.float32)
        xi = xf_ref[pl.ds(i * bn + r * _BR, _BR), :]
        o_ref[r * _BR:(r + 1) * _BR, :] = (
            jnp.dot(p0 + xi, w0t, preferred_element_type=jnp.float32)
            + jnp.dot(p1 + xi, w1t, preferred_element_type=jnp.float32)
            + bias
        )


@jax.jit
def _graph_conv(x0_bf16, x0, supports, w0t, w1t, b2d):
    n, d = x0.shape
    out = w0t.shape[1]
    bn = _R * _BR
    a_specs = [
        pl.BlockSpec((1, _BR, n), functools.partial(
            lambda i, s=0, r=0: (s, i * _R + r, 0), s=s, r=r))
        for s in range(2)
        for r in range(_R)
    ]
    return pl.pallas_call(
        _graph_conv_kernel,
        grid=(pl.cdiv(n, bn),),
        in_specs=a_specs + [
            pl.BlockSpec((n, d), lambda i: (0, 0)),
            pl.BlockSpec((n, d), lambda i: (0, 0)),
            pl.BlockSpec((d, out), lambda i: (0, 0)),
            pl.BlockSpec((d, out), lambda i: (0, 0)),
            pl.BlockSpec((1, out), lambda i: (0, 0)),
        ],
        out_specs=pl.BlockSpec((bn, out), lambda i: (i, 0)),
        out_shape=jax.ShapeDtypeStruct((n, out), jnp.float32),
        compiler_params=pltpu.CompilerParams(
            dimension_semantics=("arbitrary",),
        ),
    )(*([supports] * (2 * _R)), x0_bf16, x0, w0t, w1t, b2d)


def kernel(inputs, supports, W, b):
    bsz, n, d = inputs.shape
    s = supports.shape[0]
    out_dim = W.shape[0]
    x0 = jnp.transpose(inputs, (1, 2, 0)).reshape(n, d * bsz)
    w0t = jnp.transpose(W[:, 0::s])
    w1t = jnp.transpose(W[:, 1::s])
    b2d = b.reshape(1, out_dim)

    res = _graph_conv(x0.astype(jnp.bfloat16), x0, supports, w0t, w1t, b2d)
    return res.reshape(bsz, n, out_dim)


# speedup vs baseline: 1.3860x; 1.0020x over previous
import functools

import jax
import jax.numpy as jnp
from jax.experimental import pallas as pl
from jax.experimental.pallas import tpu as pltpu

_R = 1
_BR = 200


def _graph_conv_kernel(*refs):
    i = pl.program_id(0)
    a_refs = refs[:2 * _R]
    x_ref, xf_ref, w0t_ref, w1t_ref, b_ref, o_ref = refs[2 * _R:]
    xf32_ref = xf_ref
    w0t = w0t_ref[:]
    w1t = w1t_ref[:]
    bias = b_ref[:]
    bn = _R * _BR
    for r in range(_R):
        p0 = jax.lax.dot_general(
            a_refs[r][0], xf32_ref[:], (((1,), (0,)), ((), ())),
            precision=jax.lax.Precision.DEFAULT,
            preferred_element_type=jnp.float32)
        p1 = jax.lax.dot_general(
            a_refs[_R + r][0], xf32_ref[:], (((1,), (0,)), ((), ())),
            precision=jax.lax.Precision.DEFAULT,
            preferred_element_type=jnp.float32)
        xi = xf_ref[pl.ds(i * bn + r * _BR, _BR), :]
        o_ref[r * _BR:(r + 1) * _BR, :] = (
            jnp.dot(p0 + xi, w0t, preferred_element_type=jnp.float32)
            + jnp.dot(p1 + xi, w1t, preferred_element_type=jnp.float32)
            + bias
        )


@jax.jit
def _graph_conv(x0_bf16, x0, supports, w0t, w1t, b2d):
    n, d = x0.shape
    out = w0t.shape[1]
    bn = _R * _BR
    a_specs = [
        pl.BlockSpec((1, _BR, n), functools.partial(
            lambda i, s=0, r=0: (s, i * _R + r, 0), s=s, r=r))
        for s in range(2)
        for r in range(_R)
    ]
    return pl.pallas_call(
        _graph_conv_kernel,
        grid=(pl.cdiv(n, bn),),
        in_specs=a_specs + [
            pl.BlockSpec((n, d), lambda i: (0, 0)),
            pl.BlockSpec((n, d), lambda i: (0, 0)),
            pl.BlockSpec((d, out), lambda i: (0, 0)),
            pl.BlockSpec((d, out), lambda i: (0, 0)),
            pl.BlockSpec((1, out), lambda i: (0, 0)),
        ],
        out_specs=pl.BlockSpec((bn, out), lambda i: (i, 0)),
        out_shape=jax.ShapeDtypeStruct((n, out), jnp.float32),
        compiler_params=pltpu.CompilerParams(
            dimension_semantics=("arbitrary",),
        ),
    )(*([supports] * (2 * _R)), x0_bf16, x0, w0t, w1t, b2d)


def kernel(inputs, supports, W, b):
    bsz, n, d = inputs.shape
    s = supports.shape[0]
    out_dim = W.shape[0]
    x0 = jnp.transpose(inputs, (1, 2, 0)).reshape(n, d * bsz)
    w0t = jnp.transpose(W[:, 0::s])
    w1t = jnp.transpose(W[:, 1::s])
    b2d = b.reshape(1, out_dim)

    res = _graph_conv(x0.astype(jnp.bfloat16), x0, supports, w0t, w1t, b2d)
    return res.reshape(bsz, n, out_dim)


# speedup vs baseline: 1.4114x; 1.0184x over previous
import functools

import jax
import jax.numpy as jnp
from jax.experimental import pallas as pl
from jax.experimental.pallas import tpu as pltpu

_R = 1
_BR = 200


def _graph_conv_kernel(*refs):
    i = pl.program_id(0)
    a_refs = refs[:2 * _R]
    xf_ref, w0t_ref, w1t_ref, b_ref, o_ref = refs[2 * _R:]
    xf32_ref = xf_ref
    w0t = w0t_ref[:]
    w1t = w1t_ref[:]
    bias = b_ref[:]
    bn = _R * _BR
    for r in range(_R):
        p0 = jax.lax.dot_general(
            a_refs[r][0], xf32_ref[:], (((1,), (0,)), ((), ())),
            precision=jax.lax.Precision.DEFAULT,
            preferred_element_type=jnp.float32)
        p1 = jax.lax.dot_general(
            a_refs[_R + r][0], xf32_ref[:], (((1,), (0,)), ((), ())),
            precision=jax.lax.Precision.DEFAULT,
            preferred_element_type=jnp.float32)
        xi = xf_ref[pl.ds(i * bn + r * _BR, _BR), :]
        o_ref[r * _BR:(r + 1) * _BR, :] = (
            jnp.dot(p0 + xi, w0t, preferred_element_type=jnp.float32)
            + jnp.dot(p1 + xi, w1t, preferred_element_type=jnp.float32)
            + bias
        )


@jax.jit
def _graph_conv(x0, supports, w0t, w1t, b2d):
    n, d = x0.shape
    out = w0t.shape[1]
    bn = _R * _BR
    a_specs = [
        pl.BlockSpec((1, _BR, n), functools.partial(
            lambda i, s=0, r=0: (s, i * _R + r, 0), s=s, r=r))
        for s in range(2)
        for r in range(_R)
    ]
    return pl.pallas_call(
        _graph_conv_kernel,
        grid=(pl.cdiv(n, bn),),
        in_specs=a_specs + [
            pl.BlockSpec((n, d), lambda i: (0, 0)),
            pl.BlockSpec((d, out), lambda i: (0, 0)),
            pl.BlockSpec((d, out), lambda i: (0, 0)),
            pl.BlockSpec((1, out), lambda i: (0, 0)),
        ],
        out_specs=pl.BlockSpec((bn, out), lambda i: (i, 0)),
        out_shape=jax.ShapeDtypeStruct((n, out), jnp.float32),
        compiler_params=pltpu.CompilerParams(
            dimension_semantics=("arbitrary",),
        ),
    )(*([supports] * (2 * _R)), x0, w0t, w1t, b2d)


def kernel(inputs, supports, W, b):
    bsz, n, d = inputs.shape
    s = supports.shape[0]
    out_dim = W.shape[0]
    x0 = jnp.transpose(inputs, (1, 2, 0)).reshape(n, d * bsz)
    w0t = jnp.transpose(W[:, 0::s])
    w1t = jnp.transpose(W[:, 1::s])
    b2d = b.reshape(1, out_dim)

    res = _graph_conv(x0, supports, w0t, w1t, b2d)
    return res.reshape(bsz, n, out_dim)
